# baseline (speedup 1.0000x reference)
import jax
import jax.numpy as jnp
from jax.experimental import pallas as pl
from jax.experimental.pallas import tpu as pltpu

POS_IOU = 0.6
NEG_IOU = 0.4
FOCAL_ALPHA = 0.25
FOCAL_GAMMA = 2.0
CLS_W, REG_W, OBJ_W = 1.0, 2.0, 1.0

_A, _H, _W = 6, 200, 200
_N = _A * _H * _W
_C = 10
_G = 50
_LANES = 128
_ROWS = _N // _LANES
_BR = 32
_GRID = (_ROWS + _BR - 1) // _BR


def _loss_body(comps_ref, scores_ref, obj_ref, gt_ref, lab_ref, out_ref, acc_ref):
    i = pl.program_id(0)

    @pl.when(i == 0)
    def _init():
        for k in range(5):
            acc_ref[k] = jnp.float32(0.0)

    x = comps_ref[0]
    y = comps_ref[1]
    w = comps_ref[3]
    l = comps_ref[4]
    bx1 = x - w * 0.5
    by1 = y - l * 0.5
    bx2 = x + w * 0.5
    by2 = y + l * 0.5
    area1 = (bx2 - bx1) * (by2 - by1)

    best = jnp.full(x.shape, -jnp.inf, jnp.float32)
    asg = [jnp.zeros(x.shape, jnp.float32) for _ in range(7)]
    alab = jnp.zeros(x.shape, jnp.float32)
    for j in range(_G):
        gx = gt_ref[j, 0]
        gy = gt_ref[j, 1]
        gw = gt_ref[j, 3]
        gl = gt_ref[j, 4]
        gx1 = gx - gw * 0.5
        gy1 = gy - gl * 0.5
        gx2 = gx + gw * 0.5
        gy2 = gy + gl * 0.5
        ga = (gx2 - gx1) * (gy2 - gy1)
        ix1 = jnp.maximum(bx1, gx1)
        iy1 = jnp.maximum(by1, gy1)
        ix2 = jnp.minimum(bx2, gx2)
        iy2 = jnp.minimum(by2, gy2)
        inter = jnp.maximum(ix2 - ix1, 0.0) * jnp.maximum(iy2 - iy1, 0.0)
        union = area1 + ga - inter
        iou = inter / (union + 1e-6)
        upd = iou > best
        best = jnp.where(upd, iou, best)
        for c in range(7):
            asg[c] = jnp.where(upd, gt_ref[j, c], asg[c])
        alab = jnp.where(upd, lab_ref[j], alab)

    rows = i * _BR + jax.lax.broadcasted_iota(jnp.int32, x.shape, 0)
    inb = rows < _ROWS
    pos = (best >= POS_IOU) & inb
    neg = (best <= NEG_IOU) & inb
    valid = pos | neg

    focal_acc = jnp.zeros(x.shape, jnp.float32)
    for c in range(_C):
        s = scores_ref[c]
        t = pos & (alab == jnp.float32(c))
        nn = s >= 0.0
        e = jnp.exp(-jnp.abs(s))
        r = 1.0 / (1.0 + e)
        a = e * r
        ce = jnp.maximum(s, 0.0) - jnp.where(t, s, 0.0) + jnp.log1p(e)
        omp = jnp.where(t != nn, r, a)
        focal = jnp.where(t, FOCAL_ALPHA, 1.0 - FOCAL_ALPHA) * (omp * omp) * ce
        focal_acc = focal_acc + focal
    cls_part = jnp.sum(jnp.where(valid, focal_acc, 0.0))

    sl1_acc = jnp.zeros(x.shape, jnp.float32)
    for c in range(7):
        d = comps_ref[c] - asg[c]
        ad = jnp.abs(d)
        sl1 = jnp.where(ad < 1.0, 0.5 * d * d, ad - 0.5)
        sl1_acc = sl1_acc + sl1
    reg_part = jnp.sum(jnp.where(pos, sl1_acc, 0.0))

    o = obj_ref[...]
    posf = jnp.where(pos, 1.0, 0.0).astype(jnp.float32)
    bce = jnp.maximum(o, 0.0) - o * posf + jnp.log1p(jnp.exp(-jnp.abs(o)))
    obj_part = jnp.sum(jnp.where(valid, bce, 0.0))

    pos_part = jnp.sum(posf)
    val_part = jnp.sum(jnp.where(valid, 1.0, 0.0).astype(jnp.float32))

    acc_ref[0] = acc_ref[0] + cls_part
    acc_ref[1] = acc_ref[1] + reg_part
    acc_ref[2] = acc_ref[2] + obj_part
    acc_ref[3] = acc_ref[3] + pos_part
    acc_ref[4] = acc_ref[4] + val_part

    @pl.when(i == _GRID - 1)
    def _fin():
        cls_sum = acc_ref[0]
        reg_sum = acc_ref[1]
        obj_sum = acc_ref[2]
        pos_cnt = acc_ref[3]
        val_cnt = acc_ref[4]
        cls_loss = cls_sum / (val_cnt + jnp.float32(1e-6))
        reg_loss = jnp.where(pos_cnt > 0.0,
                             reg_sum / jnp.maximum(pos_cnt * 7.0, 1.0),
                             jnp.float32(0.0))
        obj_loss = jnp.where(val_cnt > 0.0,
                             obj_sum / jnp.maximum(val_cnt, 1.0),
                             jnp.float32(0.0))
        total = CLS_W * cls_loss + REG_W * reg_loss + OBJ_W * obj_loss
        out_ref[0] = total
        out_ref[1] = cls_loss
        out_ref[2] = reg_loss
        out_ref[3] = obj_loss


def kernel(pred_boxes, pred_scores, pred_objectness, gt_boxes, gt_labels):
    comps = pred_boxes.reshape(_N, 7).T.reshape(7, _ROWS, _LANES)
    scores = pred_scores.transpose(1, 0, 2, 3).reshape(_C, _ROWS, _LANES)
    obj = pred_objectness.reshape(_ROWS, _LANES)
    lab = gt_labels.astype(jnp.float32)

    out = pl.pallas_call(
        _loss_body,
        grid=(_GRID,),
        in_specs=[
            pl.BlockSpec((7, _BR, _LANES), lambda i: (0, i, 0)),
            pl.BlockSpec((_C, _BR, _LANES), lambda i: (0, i, 0)),
            pl.BlockSpec((_BR, _LANES), lambda i: (i, 0)),
            pl.BlockSpec(memory_space=pltpu.SMEM),
            pl.BlockSpec(memory_space=pltpu.SMEM),
        ],
        out_specs=pl.BlockSpec(memory_space=pltpu.SMEM),
        out_shape=jax.ShapeDtypeStruct((4,), jnp.float32),
        scratch_shapes=[pltpu.SMEM((5,), jnp.float32)],
    )(comps, scores, obj, gt_boxes, lab)

    return out[0], out[1], out[2], out[3]


# speedup vs baseline: 1.0209x; 1.0209x over previous
import jax
import jax.numpy as jnp
from jax.experimental import pallas as pl
from jax.experimental.pallas import tpu as pltpu

POS_IOU = 0.6
NEG_IOU = 0.4
FOCAL_ALPHA = 0.25
FOCAL_GAMMA = 2.0
CLS_W, REG_W, OBJ_W = 1.0, 2.0, 1.0

_A, _H, _W = 6, 200, 200
_N = _A * _H * _W
_C = 10
_G = 50
_LANES = 128
_ROWS = _N // _LANES
_BR = 64
_GRID = (_ROWS + _BR - 1) // _BR


def _loss_body(comps_ref, scores_ref, obj_ref, gt_ref, lab_ref, out_ref, acc_ref):
    i = pl.program_id(0)

    @pl.when(i == 0)
    def _init():
        for k in range(5):
            acc_ref[k] = jnp.float32(0.0)

    x = comps_ref[0]
    y = comps_ref[1]
    w = comps_ref[3]
    l = comps_ref[4]
    bx1 = x - w * 0.5
    by1 = y - l * 0.5
    bx2 = x + w * 0.5
    by2 = y + l * 0.5
    area1 = (bx2 - bx1) * (by2 - by1)

    best = jnp.full(x.shape, -jnp.inf, jnp.float32)
    asg = [jnp.zeros(x.shape, jnp.float32) for _ in range(7)]
    alab = jnp.zeros(x.shape, jnp.float32)
    for j in range(_G):
        gx = gt_ref[j, 0]
        gy = gt_ref[j, 1]
        gw = gt_ref[j, 3]
        gl = gt_ref[j, 4]
        gx1 = gx - gw * 0.5
        gy1 = gy - gl * 0.5
        gx2 = gx + gw * 0.5
        gy2 = gy + gl * 0.5
        ga = (gx2 - gx1) * (gy2 - gy1)
        ix1 = jnp.maximum(bx1, gx1)
        iy1 = jnp.maximum(by1, gy1)
        ix2 = jnp.minimum(bx2, gx2)
        iy2 = jnp.minimum(by2, gy2)
        inter = jnp.maximum(ix2 - ix1, 0.0) * jnp.maximum(iy2 - iy1, 0.0)
        union = area1 + ga - inter
        iou = inter / (union + 1e-6)
        upd = iou > best
        best = jnp.where(upd, iou, best)
        for c in range(7):
            asg[c] = jnp.where(upd, gt_ref[j, c], asg[c])
        alab = jnp.where(upd, lab_ref[j], alab)

    rows = i * _BR + jax.lax.broadcasted_iota(jnp.int32, x.shape, 0)
    inb = rows < _ROWS
    pos = (best >= POS_IOU) & inb
    neg = (best <= NEG_IOU) & inb
    valid = pos | neg

    focal_acc = jnp.zeros(x.shape, jnp.float32)
    for c in range(_C):
        s = scores_ref[c]
        t = pos & (alab == jnp.float32(c))
        nn = s >= 0.0
        e = jnp.exp(-jnp.abs(s))
        r = 1.0 / (1.0 + e)
        a = e * r
        ce = jnp.maximum(s, 0.0) - jnp.where(t, s, 0.0) + jnp.log1p(e)
        omp = jnp.where(t != nn, r, a)
        focal = jnp.where(t, FOCAL_ALPHA, 1.0 - FOCAL_ALPHA) * (omp * omp) * ce
        focal_acc = focal_acc + focal
    cls_part = jnp.sum(jnp.where(valid, focal_acc, 0.0))

    sl1_acc = jnp.zeros(x.shape, jnp.float32)
    for c in range(7):
        d = comps_ref[c] - asg[c]
        ad = jnp.abs(d)
        sl1 = jnp.where(ad < 1.0, 0.5 * d * d, ad - 0.5)
        sl1_acc = sl1_acc + sl1
    reg_part = jnp.sum(jnp.where(pos, sl1_acc, 0.0))

    o = obj_ref[...]
    posf = jnp.where(pos, 1.0, 0.0).astype(jnp.float32)
    bce = jnp.maximum(o, 0.0) - o * posf + jnp.log1p(jnp.exp(-jnp.abs(o)))
    obj_part = jnp.sum(jnp.where(valid, bce, 0.0))

    pos_part = jnp.sum(posf)
    val_part = jnp.sum(jnp.where(valid, 1.0, 0.0).astype(jnp.float32))

    acc_ref[0] = acc_ref[0] + cls_part
    acc_ref[1] = acc_ref[1] + reg_part
    acc_ref[2] = acc_ref[2] + obj_part
    acc_ref[3] = acc_ref[3] + pos_part
    acc_ref[4] = acc_ref[4] + val_part

    @pl.when(i == _GRID - 1)
    def _fin():
        cls_sum = acc_ref[0]
        reg_sum = acc_ref[1]
        obj_sum = acc_ref[2]
        pos_cnt = acc_ref[3]
        val_cnt = acc_ref[4]
        cls_loss = cls_sum / (val_cnt + jnp.float32(1e-6))
        reg_loss = jnp.where(pos_cnt > 0.0,
                             reg_sum / jnp.maximum(pos_cnt * 7.0, 1.0),
                             jnp.float32(0.0))
        obj_loss = jnp.where(val_cnt > 0.0,
                             obj_sum / jnp.maximum(val_cnt, 1.0),
                             jnp.float32(0.0))
        total = CLS_W * cls_loss + REG_W * reg_loss + OBJ_W * obj_loss
        out_ref[0] = total
        out_ref[1] = cls_loss
        out_ref[2] = reg_loss
        out_ref[3] = obj_loss


def kernel(pred_boxes, pred_scores, pred_objectness, gt_boxes, gt_labels):
    comps = pred_boxes.reshape(_N, 7).T.reshape(7, _ROWS, _LANES)
    scores = pred_scores.transpose(1, 0, 2, 3).reshape(_C, _ROWS, _LANES)
    obj = pred_objectness.reshape(_ROWS, _LANES)
    lab = gt_labels.astype(jnp.float32)

    out = pl.pallas_call(
        _loss_body,
        grid=(_GRID,),
        in_specs=[
            pl.BlockSpec((7, _BR, _LANES), lambda i: (0, i, 0)),
            pl.BlockSpec((_C, _BR, _LANES), lambda i: (0, i, 0)),
            pl.BlockSpec((_BR, _LANES), lambda i: (i, 0)),
            pl.BlockSpec(memory_space=pltpu.SMEM),
            pl.BlockSpec(memory_space=pltpu.SMEM),
        ],
        out_specs=pl.BlockSpec(memory_space=pltpu.SMEM),
        out_shape=jax.ShapeDtypeStruct((4,), jnp.float32),
        scratch_shapes=[pltpu.SMEM((5,), jnp.float32)],
    )(comps, scores, obj, gt_boxes, lab)

    return out[0], out[1], out[2], out[3]
